# Initial kernel scaffold; baseline (speedup 1.0000x reference)
import jax
import jax.numpy as jnp
from jax.experimental import pallas as pl


def kernel(x, batch_ids, W1, b1, W2, b2):
    raise NotImplementedError("write your pallas kernel here")


# speedup vs baseline: 18.4843x; 18.4843x over previous
import functools

import jax
import jax.numpy as jnp
from jax.experimental import pallas as pl
from jax.experimental.pallas import tpu as pltpu

N = 50000
D = 512
G = 128
H = 256
BN = 2000
STEPS = N // BN


def _psrnet_kernel(ids_ref, x_ref, w1_ref, b1_ref, w2_ref, b2_ref,
                   out_ref, sums_ref, counts_ref):
    step = pl.program_id(0)

    @pl.when(step == 0)
    def _init():
        sums_ref[:, :] = jnp.zeros_like(sums_ref)
        counts_ref[:, :] = jnp.zeros_like(counts_ref)

    ids = ids_ref[0, 0, :]
    onehot = (ids[None, :] == jax.lax.broadcasted_iota(jnp.int32, (G, BN), 0)
              ).astype(jnp.float32)
    sums_ref[:, :] += jnp.dot(onehot, x_ref[:, :],
                              preferred_element_type=jnp.float32)
    counts_ref[0, :] += jnp.sum(onehot, axis=1)

    @pl.when(step == STEPS - 1)
    def _finish():
        mean = sums_ref[:, :] / jnp.maximum(counts_ref[0, :], 1.0)[:, None]
        h = jnp.maximum(
            jnp.dot(mean, w1_ref[:, :], preferred_element_type=jnp.float32)
            + b1_ref[0, :][None, :], 0.0)
        out_ref[:, :] = (jnp.dot(h, w2_ref[:, :],
                                 preferred_element_type=jnp.float32)
                         + b2_ref[0, :][None, :])


@jax.jit
def kernel(x, batch_ids, W1, b1, W2, b2):
    ids3 = batch_ids.astype(jnp.int32).reshape(STEPS, 1, BN)
    W2p = jnp.pad(W2, ((0, 0), (0, 127)))
    b2p = jnp.pad(b2, (0, 127)).reshape(1, 128)
    b1r = b1.reshape(1, H)

    out = pl.pallas_call(
        _psrnet_kernel,
        grid=(STEPS,),
        in_specs=[
            pl.BlockSpec((1, 1, BN), lambda i: (i, 0, 0)),
            pl.BlockSpec((BN, D), lambda i: (i, 0)),
            pl.BlockSpec((D, H), lambda i: (0, 0)),
            pl.BlockSpec((1, H), lambda i: (0, 0)),
            pl.BlockSpec((H, 128), lambda i: (0, 0)),
            pl.BlockSpec((1, 128), lambda i: (0, 0)),
        ],
        out_specs=pl.BlockSpec((G, 128), lambda i: (0, 0)),
        out_shape=jax.ShapeDtypeStruct((G, 128), jnp.float32),
        scratch_shapes=[
            pltpu.VMEM((G, D), jnp.float32),
            pltpu.VMEM((8, G), jnp.float32),
        ],
    )(ids3, x, W1, b1r, W2p, b2p)
    return out[:, :1]
